# Initial kernel scaffold; baseline (speedup 1.0000x reference)
import jax
import jax.numpy as jnp
from jax.experimental import pallas as pl


def kernel(sensory_input, W_in, b_in, W_ff, b_ff, W_fb, b_fb, proj_W, proj_b, som_weights):
    raise NotImplementedError("write your pallas kernel here")


# speedup vs baseline: 1.9195x; 1.9195x over previous
import functools

import jax
import jax.numpy as jnp
from jax import lax
from jax.experimental import pallas as pl
from jax.experimental.pallas import tpu as pltpu

MAP_H, MAP_W = 64, 64
FEATURE_DIM = 256
NUM_NEURONS = 16384
BATCH = 1024
THRESHOLD = 1.0
LR = 0.005
A_PLUS = 1.0
SIGMA = 2.0
K = MAP_H * MAP_W

COL_TILE = 1024
N_TILES = NUM_NEURONS // COL_TILE


def _front_kernel(x_blk, w_in_blk, b_in, w_ff, b_ff, proj_w, proj_b,
                  feat_out, act_out, acc):
    j = pl.program_id(0)

    @pl.when(j == 0)
    def _():
        acc[...] = jnp.zeros_like(acc)

    colsum = jnp.sum(x_blk[...], axis=0, keepdims=True)
    acc[...] += lax.dot_general(
        colsum, w_in_blk[...], (((1,), (1,)), ((), ())),
        preferred_element_type=jnp.float32)

    @pl.when(j == N_TILES - 1)
    def _():
        i_in = acc[...] * (1.0 / BATCH) + b_in[...]
        v = i_in
        spikes = jax.nn.sigmoid((v - THRESHOLD) * 2.0)
        v_reset = v - spikes * THRESHOLD
        out_ff = lax.dot_general(
            spikes, w_ff[...], (((1,), (1,)), ((), ())),
            preferred_element_type=jnp.float32) + b_ff[...]
        feat = lax.dot_general(
            out_ff, proj_w[...], (((1,), (1,)), ((), ())),
            preferred_element_type=jnp.float32) + proj_b[...]
        feat_out[...] = jnp.maximum(feat, 0.0)
        act_out[...] = (jnp.mean(v_reset, keepdims=True)
                        + jnp.mean(spikes, keepdims=True)).reshape(1, 1) * 0.5


def _som_kernel(feat, som, s_out):
    x = feat[...]
    w = som[...]
    ones = jnp.ones((1, FEATURE_DIM), jnp.float32)
    norms = lax.dot_general(ones, w * w, (((1,), (1,)), ((), ())),
                            preferred_element_type=jnp.float32)
    dots = lax.dot_general(x, w, (((1,), (1,)), ((), ())),
                           preferred_element_type=jnp.float32)
    xnorm = jnp.sum(x * x)
    d = norms - 2.0 * dots + xnorm

    k_idx = lax.broadcasted_iota(jnp.int32, (1, K), 1)
    r = k_idx >> 6
    c = k_idx & 63

    s = None
    for t in range(4):
        m = jnp.min(d, axis=1, keepdims=True)
        cand = jnp.where(d <= m, k_idx, K)
        bmu = jnp.min(cand, axis=1, keepdims=True)
        br = bmu >> 6
        bc = bmu & 63
        gd2 = ((r - br) * (r - br) + (c - bc) * (c - bc)).astype(jnp.float32)
        s = jnp.exp(gd2 * (-1.0 / (2.0 * SIGMA * SIGMA)))
        if t < 3:
            f = 1.0 - (LR * A_PLUS) * s
            d = d * f * f
    s_out[...] = s


def kernel(sensory_input, W_in, b_in, W_ff, b_ff, W_fb, b_fb, proj_W, proj_b,
           som_weights):
    del W_fb, b_fb
    feat, act = pl.pallas_call(
        _front_kernel,
        grid=(N_TILES,),
        in_specs=[
            pl.BlockSpec((BATCH, COL_TILE), lambda j: (0, j)),
            pl.BlockSpec((FEATURE_DIM, COL_TILE), lambda j: (0, j)),
            pl.BlockSpec((1, FEATURE_DIM), lambda j: (0, 0)),
            pl.BlockSpec((FEATURE_DIM, FEATURE_DIM), lambda j: (0, 0)),
            pl.BlockSpec((1, FEATURE_DIM), lambda j: (0, 0)),
            pl.BlockSpec((FEATURE_DIM, FEATURE_DIM), lambda j: (0, 0)),
            pl.BlockSpec((1, FEATURE_DIM), lambda j: (0, 0)),
        ],
        out_specs=[
            pl.BlockSpec((1, FEATURE_DIM), lambda j: (0, 0)),
            pl.BlockSpec((1, 1), lambda j: (0, 0)),
        ],
        out_shape=[
            jax.ShapeDtypeStruct((1, FEATURE_DIM), jnp.float32),
            jax.ShapeDtypeStruct((1, 1), jnp.float32),
        ],
        scratch_shapes=[pltpu.VMEM((1, FEATURE_DIM), jnp.float32)],
    )(sensory_input, W_in, b_in.reshape(1, -1), W_ff, b_ff.reshape(1, -1),
      proj_W, proj_b.reshape(1, -1))

    s = pl.pallas_call(
        _som_kernel,
        out_shape=jax.ShapeDtypeStruct((1, K), jnp.float32),
    )(feat, som_weights)

    return s.reshape(K), act.reshape(())


# speedup vs baseline: 2.0049x; 1.0445x over previous
import jax
import jax.numpy as jnp
from jax import lax
from jax.experimental import pallas as pl
from jax.experimental.pallas import tpu as pltpu

MAP_H, MAP_W = 64, 64
FEATURE_DIM = 256
NUM_NEURONS = 16384
BATCH = 1024
THRESHOLD = 1.0
LR = 0.005
A_PLUS = 1.0
SIGMA = 2.0
K = MAP_H * MAP_W

COL_TILE = 1024
N_TILES = NUM_NEURONS // COL_TILE


def _fused_kernel(x_blk, w_in_blk, b_in, w_ff, b_ff, proj_w, proj_b, som,
                  s_out, act_out, acc, norms):
    j = pl.program_id(0)
    ones_d = jnp.ones((1, FEATURE_DIM), jnp.float32)

    @pl.when(j == 0)
    def _():
        acc[...] = jnp.zeros_like(acc)
        w = som[...]
        norms[...] = lax.dot_general(ones_d, w * w, (((1,), (1,)), ((), ())),
                                     preferred_element_type=jnp.float32)

    colsum = jnp.sum(x_blk[...], axis=0, keepdims=True)
    acc[...] += lax.dot_general(
        colsum, w_in_blk[...], (((1,), (1,)), ((), ())),
        preferred_element_type=jnp.float32)

    @pl.when(j == N_TILES - 1)
    def _():
        i_in = acc[...] * (1.0 / BATCH) + b_in[...]
        v = i_in
        spikes = jax.nn.sigmoid((v - THRESHOLD) * 2.0)
        v_reset = v - spikes * THRESHOLD
        out_ff = lax.dot_general(
            spikes, w_ff[...], (((1,), (1,)), ((), ())),
            preferred_element_type=jnp.float32) + b_ff[...]
        feat = lax.dot_general(
            out_ff, proj_w[...], (((1,), (1,)), ((), ())),
            preferred_element_type=jnp.float32) + proj_b[...]
        x = jnp.maximum(feat, 0.0)
        act_out[...] = (jnp.mean(v_reset, keepdims=True)
                        + jnp.mean(spikes, keepdims=True)).reshape(1, 1) * 0.5

        w = som[...]
        dots = lax.dot_general(x, w, (((1,), (1,)), ((), ())),
                               preferred_element_type=jnp.float32)
        d = norms[...] - 2.0 * dots + jnp.sum(x * x)

        k_idx = lax.broadcasted_iota(jnp.int32, (1, K), 1)
        r = k_idx >> 6
        c = k_idx & 63
        s = None
        for t in range(4):
            m = jnp.min(d, axis=1, keepdims=True)
            cand = jnp.where(d <= m, k_idx, K)
            bmu = jnp.min(cand, axis=1, keepdims=True)
            br = bmu >> 6
            bc = bmu & 63
            gd2 = ((r - br) * (r - br) + (c - bc) * (c - bc)).astype(jnp.float32)
            s = jnp.exp(gd2 * (-1.0 / (2.0 * SIGMA * SIGMA)))
            if t < 3:
                f = 1.0 - (LR * A_PLUS) * s
                d = d * f * f
        s_out[...] = s


def kernel(sensory_input, W_in, b_in, W_ff, b_ff, W_fb, b_fb, proj_W, proj_b,
           som_weights):
    del W_fb, b_fb
    s, act = pl.pallas_call(
        _fused_kernel,
        grid=(N_TILES,),
        in_specs=[
            pl.BlockSpec((BATCH, COL_TILE), lambda j: (0, j)),
            pl.BlockSpec((FEATURE_DIM, COL_TILE), lambda j: (0, j)),
            pl.BlockSpec((1, FEATURE_DIM), lambda j: (0, 0)),
            pl.BlockSpec((FEATURE_DIM, FEATURE_DIM), lambda j: (0, 0)),
            pl.BlockSpec((1, FEATURE_DIM), lambda j: (0, 0)),
            pl.BlockSpec((FEATURE_DIM, FEATURE_DIM), lambda j: (0, 0)),
            pl.BlockSpec((1, FEATURE_DIM), lambda j: (0, 0)),
            pl.BlockSpec((K, FEATURE_DIM), lambda j: (0, 0)),
        ],
        out_specs=[
            pl.BlockSpec((1, K), lambda j: (0, 0)),
            pl.BlockSpec((1, 1), lambda j: (0, 0)),
        ],
        out_shape=[
            jax.ShapeDtypeStruct((1, K), jnp.float32),
            jax.ShapeDtypeStruct((1, 1), jnp.float32),
        ],
        scratch_shapes=[
            pltpu.VMEM((1, FEATURE_DIM), jnp.float32),
            pltpu.VMEM((1, K), jnp.float32),
        ],
    )(sensory_input, W_in, b_in.reshape(1, -1), W_ff, b_ff.reshape(1, -1),
      proj_W, proj_b.reshape(1, -1), som_weights)

    return s.reshape(K), act.reshape(())
